# Initial kernel scaffold; baseline (speedup 1.0000x reference)
import jax
import jax.numpy as jnp
from jax.experimental import pallas as pl


def kernel(x_rid, x_cell, src_fwd, dst_fwd, src_bwd, dst_bwd, W1_c, b1_c, W1_i, b1_i, W2_c, b2_c, W2_i, b2_i):
    raise NotImplementedError("write your pallas kernel here")


# speedup vs baseline: 1.3516x; 1.3516x over previous
import jax
import jax.numpy as jnp
from jax import lax
from jax.experimental import pallas as pl
from jax.experimental.pallas import tpu as pltpu
from jax.experimental.pallas import tpu_sc as plsc

N_RID_C = 10000
N_CELL_C = 50000
E_C = 250000
D = 128

NW = 32
EPT = 7936
E_PAD = NW * EPT
EC = 64
TRIPS = EPT // EC
ZB = 16

Q_CELL, NP_CELL = 12544, 4
Q_RID, NP_RID = 10752, 1
MP_CELL = Q_CELL * NP_CELL
MP_RID = Q_RID * NP_RID


def _make_sc_agg(Q, NP, ones_mode=False):
    zone = Q // 16
    assert zone % ZB == 0
    mesh = plsc.VectorSubcoreMesh(core_axis_name="c", subcore_axis_name="s")
    out_type = jax.ShapeDtypeStruct((2, NP * Q, D), jnp.float32)
    scratch = [
        pltpu.VMEM((EC,), jnp.int32),
        pltpu.VMEM((TRIPS, EC), jnp.int32),
        pltpu.VMEM((EC, D), jnp.float32),
        pltpu.VMEM((ZB, D), jnp.float32),
        pltpu.VMEM_SHARED((Q + 16, D), jnp.float32),
        pltpu.SemaphoreType.DMA,
    ]

    def body(*refs):
        if ones_mode:
            (src_hbm, dst_hbm, agg_out, sidx, cldst, rowbuf, zrow, acc_sh,
             sem) = refs
            x_hbm = None
        else:
            (x_hbm, src_hbm, dst_hbm, agg_out, sidx, cldst, rowbuf, zrow,
             acc_sh, sem) = refs
        cid = lax.axis_index("c")
        sid = lax.axis_index("s")
        wid = sid * 2 + cid
        z0 = sid * zone

        zero16f = jnp.zeros((16,), jnp.float32)

        def zrow_init(r, c2):
            for c in range(D // 16):
                zrow[r, pl.ds(c * 16, 16)] = zero16f
            return c2
        lax.fori_loop(0, ZB, zrow_init, 0)

        if ones_mode:
            one16f = jnp.ones((16,), jnp.float32)

            def ones_init(r, c2):
                for c in range(D // 16):
                    rowbuf[r, pl.ds(c * 16, 16)] = one16f
                return c2
            lax.fori_loop(0, EC, ones_init, 0)

        garb = jnp.full((16,), Q, jnp.int32) + lax.iota(jnp.int32, 16)

        for p in range(NP):
            lo = p * Q

            def zacc(r, c2):
                pltpu.sync_copy(zrow, acc_sh.at[pl.ds(z0 + r * ZB, ZB)])
                return c2
            lax.fori_loop(0, zone // ZB, zacc, 0)

            def bld(t, c2):
                pltpu.sync_copy(dst_hbm.at[pl.ds(wid * EPT + t * EC, EC)],
                                cldst.at[t])
                for k in range(EC // 16):
                    dv = cldst[t, pl.ds(k * 16, 16)]
                    m = (dv >= lo) & (dv < lo + Q)
                    cldst[t, pl.ds(k * 16, 16)] = jnp.where(m, dv - lo, garb)
                return c2
            lax.fori_loop(0, TRIPS, bld, 0)

            plsc.subcore_barrier()

            def gs(t, c2):
                if not ones_mode:
                    pltpu.sync_copy(
                        src_hbm.at[pl.ds(wid * EPT + t * EC, EC)], sidx)
                    pltpu.async_copy(x_hbm.at[sidx], rowbuf, sem).wait()
                pltpu.sync_copy(rowbuf, acc_sh.at[cldst.at[t]], add=True)
                return c2
            lax.fori_loop(0, TRIPS, gs, 0)

            plsc.subcore_barrier()

            pltpu.sync_copy(acc_sh.at[pl.ds(z0, zone)],
                            agg_out.at[cid, pl.ds(lo + z0, zone)])

    return pl.kernel(body, out_type=out_type, mesh=mesh,
                     scratch_types=tuple(scratch))


def _sc_agg(x, src_p, dst_p, Q, NP):
    return _make_sc_agg(Q, NP)(x, src_p, dst_p)


def _sc_deg(src_p, dst_p, Q, NP):
    return _make_sc_agg(Q, NP, ones_mode=True)(src_p, dst_p)


def _dense1(aggP, degP, x_pad, W):
    Mp = x_pad.shape[0]
    R = 512
    nb = Mp // R

    def body(a_ref, d_ref, x_ref, w_ref, h_ref, cs_ref, acc):
        i = pl.program_id(0)
        a = a_ref[0] + a_ref[1]
        dgr = d_ref[0] + d_ref[1]
        rinv = 1.0 / (dgr + 1.0)
        hn = (a + x_ref[...]) * rinv
        h_ref[...] = jnp.dot(hn, w_ref[...],
                             preferred_element_type=jnp.float32)

        @pl.when(i == 0)
        def _():
            acc[...] = jnp.zeros_like(acc)
        acc[...] += jnp.sum(hn, axis=0, keepdims=True)
        cs_ref[...] = acc[...]

    return pl.pallas_call(
        body,
        grid=(nb,),
        in_specs=[
            pl.BlockSpec((2, R, D), lambda i: (0, i, 0)),
            pl.BlockSpec((2, R, 1), lambda i: (0, i, 0)),
            pl.BlockSpec((R, D), lambda i: (i, 0)),
            pl.BlockSpec((D, D), lambda i: (0, 0)),
        ],
        out_specs=[
            pl.BlockSpec((R, D), lambda i: (i, 0)),
            pl.BlockSpec((1, D), lambda i: (0, 0)),
        ],
        out_shape=[
            jax.ShapeDtypeStruct((Mp, D), jnp.float32),
            jax.ShapeDtypeStruct((1, D), jnp.float32),
        ],
        scratch_shapes=[pltpu.VMEM((1, D), jnp.float32)],
    )(aggP, degP, x_pad, W)


def _dense2(agg2P, degP, h_unc, cs_own, cs_src, W1_own, W1_src, W2, n_own,
            n_src):
    Mp = h_unc.shape[0]
    R = 512
    nb = Mp // R

    def body(a_ref, d_ref, h_ref, cso_ref, css_ref, w1o_ref, w1s_ref,
             w2_ref, h2_ref, cs2_ref, acc):
        i = pl.program_id(0)
        c_own = jnp.dot(cso_ref[...] * (1.0 / n_own), w1o_ref[...],
                        preferred_element_type=jnp.float32)
        c_src = jnp.dot(css_ref[...] * (1.0 / n_src), w1s_ref[...],
                        preferred_element_type=jnp.float32)
        a = a_ref[0] + a_ref[1]
        dgr = d_ref[0] + d_ref[1]
        rinv = 1.0 / (dgr + 1.0)
        hn2 = (a - dgr * c_src + h_ref[...] - c_own) * rinv
        ridx = i * R + lax.broadcasted_iota(jnp.int32, (R, 1), 0)
        hn2 = jnp.where(ridx < n_own, hn2, 0.0)
        h2_ref[...] = jnp.dot(hn2, w2_ref[...],
                              preferred_element_type=jnp.float32)

        @pl.when(i == 0)
        def _():
            acc[...] = jnp.zeros_like(acc)
        acc[...] += jnp.sum(hn2, axis=0, keepdims=True)
        cs2_ref[...] = acc[...]

    return pl.pallas_call(
        body,
        grid=(nb,),
        in_specs=[
            pl.BlockSpec((2, R, D), lambda i: (0, i, 0)),
            pl.BlockSpec((2, R, 1), lambda i: (0, i, 0)),
            pl.BlockSpec((R, D), lambda i: (i, 0)),
            pl.BlockSpec((1, D), lambda i: (0, 0)),
            pl.BlockSpec((1, D), lambda i: (0, 0)),
            pl.BlockSpec((D, D), lambda i: (0, 0)),
            pl.BlockSpec((D, D), lambda i: (0, 0)),
            pl.BlockSpec((D, D), lambda i: (0, 0)),
        ],
        out_specs=[
            pl.BlockSpec((R, D), lambda i: (i, 0)),
            pl.BlockSpec((1, D), lambda i: (0, 0)),
        ],
        out_shape=[
            jax.ShapeDtypeStruct((Mp, D), jnp.float32),
            jax.ShapeDtypeStruct((1, D), jnp.float32),
        ],
        scratch_shapes=[pltpu.VMEM((1, D), jnp.float32)],
    )(agg2P, degP, h_unc, cs_own, cs_src, W1_own, W1_src, W2)


def _final(h2r, h2c, cs2_r, cs2_c, W2_i, W2_c):
    B = 400
    nb = N_CELL_C // B

    def body(hr_ref, hc_ref, csr_ref, csc_ref, wr_ref, wc_ref, o_ref):
        i = pl.program_id(0)
        c2r = jnp.dot(csr_ref[...] * (1.0 / N_RID_C), wr_ref[...],
                      preferred_element_type=jnp.float32)
        c2c = jnp.dot(csc_ref[...] * (1.0 / N_CELL_C), wc_ref[...],
                      preferred_element_type=jnp.float32)

        @pl.when(i < 25)
        def _():
            o_ref[...] = jnp.maximum(hr_ref[...] - c2r, 0.0)

        @pl.when(i >= 25)
        def _():
            o_ref[...] = jnp.maximum(hc_ref[...] - c2c, 0.0)

    return pl.pallas_call(
        body,
        grid=(nb,),
        in_specs=[
            pl.BlockSpec((B, D), lambda i: (jnp.minimum(i, 24), 0)),
            pl.BlockSpec((B, D), lambda i: (i, 0)),
            pl.BlockSpec((1, D), lambda i: (0, 0)),
            pl.BlockSpec((1, D), lambda i: (0, 0)),
            pl.BlockSpec((D, D), lambda i: (0, 0)),
            pl.BlockSpec((D, D), lambda i: (0, 0)),
        ],
        out_specs=pl.BlockSpec((B, D), lambda i: (i, 0)),
        out_shape=jax.ShapeDtypeStruct((N_CELL_C, D), jnp.float32),
    )(h2r, h2c, cs2_r, cs2_c, W2_i, W2_c)


def kernel(x_rid, x_cell, src_fwd, dst_fwd, src_bwd, dst_bwd,
           W1_c, b1_c, W1_i, b1_i, W2_c, b2_c, W2_i, b2_i):
    f32 = jnp.float32
    pad_e = E_PAD - E_C
    srcf = jnp.concatenate([src_fwd, jnp.zeros((pad_e,), jnp.int32)])
    dstf = jnp.concatenate([dst_fwd, jnp.full((pad_e,), MP_CELL, jnp.int32)])
    srcb = jnp.concatenate([src_bwd, jnp.zeros((pad_e,), jnp.int32)])
    dstb = jnp.concatenate([dst_bwd, jnp.full((pad_e,), MP_RID, jnp.int32)])
    x_cell_p = jnp.concatenate(
        [x_cell, jnp.zeros((MP_CELL - N_CELL_C, D), f32)])
    x_rid_p = jnp.concatenate(
        [x_rid, jnp.zeros((MP_RID - N_RID_C, D), f32)])

    degc_w = _sc_deg(srcf, dstf, Q_CELL, NP_CELL)
    degr_w = _sc_deg(srcb, dstb, Q_RID, NP_RID)
    degc = degc_w[:, :, 0:1]
    degr = degr_w[:, :, 0:1]

    aggc_p = _sc_agg(x_rid, srcf, dstf, Q_CELL, NP_CELL)
    aggr_p = _sc_agg(x_cell, srcb, dstb, Q_RID, NP_RID)

    h_c, cs_c = _dense1(aggc_p, degc, x_cell_p, W1_c)
    h_r, cs_r = _dense1(aggr_p, degr, x_rid_p, W1_i)

    agg2c_p = _sc_agg(h_r, srcf, dstf, Q_CELL, NP_CELL)
    agg2r_p = _sc_agg(h_c, srcb, dstb, Q_RID, NP_RID)

    h2c, cs2_c = _dense2(agg2c_p, degc, h_c, cs_c, cs_r,
                         W1_c, W1_i, W2_c, N_CELL_C, N_RID_C)
    h2r, cs2_r = _dense2(agg2r_p, degr, h_r, cs_r, cs_c,
                         W1_i, W1_c, W2_i, N_RID_C, N_CELL_C)

    return _final(h2r, h2c, cs2_r, cs2_c, W2_i, W2_c)


# speedup vs baseline: 1.3989x; 1.0350x over previous
import jax
import jax.numpy as jnp
from jax import lax
from jax.experimental import pallas as pl
from jax.experimental.pallas import tpu as pltpu
from jax.experimental.pallas import tpu_sc as plsc

N_RID_C = 10000
N_CELL_C = 50000
E_C = 250000
D = 128

NW = 32
EPT = 7936
E_PAD = NW * EPT
EC = 64
TRIPS = EPT // EC
ZB = 16

Q_CELL, NP_CELL = 12544, 4
Q_RID, NP_RID = 10752, 1
MP_CELL = Q_CELL * NP_CELL
MP_RID = Q_RID * NP_RID


def _make_sc_agg(Q, NP, ones_mode=False, ec=EC):
    zone = Q // 16
    assert zone % ZB == 0
    mesh = plsc.VectorSubcoreMesh(core_axis_name="c", subcore_axis_name="s")
    out_type = jax.ShapeDtypeStruct((2, NP * Q, D), jnp.float32)
    scratch = [
        pltpu.VMEM((ec,), jnp.int32),
        pltpu.VMEM((EPT // ec, ec), jnp.int32),
        pltpu.VMEM((ec, D), jnp.float32),
        pltpu.VMEM((ZB, D), jnp.float32),
        pltpu.VMEM_SHARED((Q + 16, D), jnp.float32),
        pltpu.SemaphoreType.DMA,
    ]

    def body(*refs):
        if ones_mode:
            (src_hbm, dst_hbm, agg_out, sidx, cldst, rowbuf, zrow, acc_sh,
             sem) = refs
            x_hbm = None
        else:
            (x_hbm, src_hbm, dst_hbm, agg_out, sidx, cldst, rowbuf, zrow,
             acc_sh, sem) = refs
        cid = lax.axis_index("c")
        sid = lax.axis_index("s")
        wid = sid * 2 + cid
        z0 = sid * zone

        zero16f = jnp.zeros((16,), jnp.float32)

        def zrow_init(r, c2):
            for c in range(D // 16):
                zrow[r, pl.ds(c * 16, 16)] = zero16f
            return c2
        lax.fori_loop(0, ZB, zrow_init, 0)

        if ones_mode:
            one16f = jnp.ones((16,), jnp.float32)

            def ones_init(r, c2):
                for c in range(D // 16):
                    rowbuf[r, pl.ds(c * 16, 16)] = one16f
                return c2
            lax.fori_loop(0, ec, ones_init, 0)

        garb = jnp.full((16,), Q, jnp.int32) + lax.iota(jnp.int32, 16)

        for p in range(NP):
            lo = p * Q

            def zacc(r, c2):
                pltpu.sync_copy(zrow, acc_sh.at[pl.ds(z0 + r * ZB, ZB)])
                return c2
            lax.fori_loop(0, zone // ZB, zacc, 0)

            def bld(t, c2):
                pltpu.sync_copy(dst_hbm.at[pl.ds(wid * EPT + t * ec, ec)],
                                cldst.at[t])
                for k in range(ec // 16):
                    dv = cldst[t, pl.ds(k * 16, 16)]
                    m = (dv >= lo) & (dv < lo + Q)
                    cldst[t, pl.ds(k * 16, 16)] = jnp.where(m, dv - lo, garb)
                return c2
            lax.fori_loop(0, EPT // ec, bld, 0)

            plsc.subcore_barrier()

            def gs(t, c2):
                if not ones_mode:
                    pltpu.sync_copy(
                        src_hbm.at[pl.ds(wid * EPT + t * ec, ec)], sidx)
                    pltpu.async_copy(x_hbm.at[sidx], rowbuf, sem).wait()
                pltpu.sync_copy(rowbuf, acc_sh.at[cldst.at[t]], add=True)
                return c2
            lax.fori_loop(0, EPT // ec, gs, 0)

            plsc.subcore_barrier()

            pltpu.sync_copy(acc_sh.at[pl.ds(z0, zone)],
                            agg_out.at[cid, pl.ds(lo + z0, zone)])

    return pl.kernel(body, out_type=out_type, mesh=mesh,
                     scratch_types=tuple(scratch))


def _sc_agg(x, src_p, dst_p, Q, NP):
    ec = 64 if NP > 1 else 128
    return _make_sc_agg(Q, NP, ec=ec)(x, src_p, dst_p)


def _sc_deg(src_p, dst_p, Q, NP):
    ec = 64 if NP > 1 else 128
    return _make_sc_agg(Q, NP, ones_mode=True, ec=ec)(src_p, dst_p)


def _dense1(aggP, degP, x_pad, W):
    Mp = x_pad.shape[0]
    R = 512
    nb = Mp // R

    def body(a_ref, d_ref, x_ref, w_ref, h_ref, cs_ref, acc):
        i = pl.program_id(0)
        a = a_ref[0] + a_ref[1]
        dgr = d_ref[0] + d_ref[1]
        rinv = 1.0 / (dgr + 1.0)
        hn = (a + x_ref[...]) * rinv
        h_ref[...] = jnp.dot(hn, w_ref[...],
                             preferred_element_type=jnp.float32)

        @pl.when(i == 0)
        def _():
            acc[...] = jnp.zeros_like(acc)
        acc[...] += jnp.sum(hn, axis=0, keepdims=True)
        cs_ref[...] = acc[...]

    return pl.pallas_call(
        body,
        grid=(nb,),
        in_specs=[
            pl.BlockSpec((2, R, D), lambda i: (0, i, 0)),
            pl.BlockSpec((2, R, 1), lambda i: (0, i, 0)),
            pl.BlockSpec((R, D), lambda i: (i, 0)),
            pl.BlockSpec((D, D), lambda i: (0, 0)),
        ],
        out_specs=[
            pl.BlockSpec((R, D), lambda i: (i, 0)),
            pl.BlockSpec((1, D), lambda i: (0, 0)),
        ],
        out_shape=[
            jax.ShapeDtypeStruct((Mp, D), jnp.float32),
            jax.ShapeDtypeStruct((1, D), jnp.float32),
        ],
        scratch_shapes=[pltpu.VMEM((1, D), jnp.float32)],
    )(aggP, degP, x_pad, W)


def _dense2(agg2P, degP, h_unc, cs_own, cs_src, W1_own, W1_src, W2, n_own,
            n_src):
    Mp = h_unc.shape[0]
    R = 512
    nb = Mp // R

    def body(a_ref, d_ref, h_ref, cso_ref, css_ref, w1o_ref, w1s_ref,
             w2_ref, h2_ref, cs2_ref, acc):
        i = pl.program_id(0)
        c_own = jnp.dot(cso_ref[...] * (1.0 / n_own), w1o_ref[...],
                        preferred_element_type=jnp.float32)
        c_src = jnp.dot(css_ref[...] * (1.0 / n_src), w1s_ref[...],
                        preferred_element_type=jnp.float32)
        a = a_ref[0] + a_ref[1]
        dgr = d_ref[0] + d_ref[1]
        rinv = 1.0 / (dgr + 1.0)
        hn2 = (a - dgr * c_src + h_ref[...] - c_own) * rinv
        ridx = i * R + lax.broadcasted_iota(jnp.int32, (R, 1), 0)
        hn2 = jnp.where(ridx < n_own, hn2, 0.0)
        h2_ref[...] = jnp.dot(hn2, w2_ref[...],
                              preferred_element_type=jnp.float32)

        @pl.when(i == 0)
        def _():
            acc[...] = jnp.zeros_like(acc)
        acc[...] += jnp.sum(hn2, axis=0, keepdims=True)
        cs2_ref[...] = acc[...]

    return pl.pallas_call(
        body,
        grid=(nb,),
        in_specs=[
            pl.BlockSpec((2, R, D), lambda i: (0, i, 0)),
            pl.BlockSpec((2, R, 1), lambda i: (0, i, 0)),
            pl.BlockSpec((R, D), lambda i: (i, 0)),
            pl.BlockSpec((1, D), lambda i: (0, 0)),
            pl.BlockSpec((1, D), lambda i: (0, 0)),
            pl.BlockSpec((D, D), lambda i: (0, 0)),
            pl.BlockSpec((D, D), lambda i: (0, 0)),
            pl.BlockSpec((D, D), lambda i: (0, 0)),
        ],
        out_specs=[
            pl.BlockSpec((R, D), lambda i: (i, 0)),
            pl.BlockSpec((1, D), lambda i: (0, 0)),
        ],
        out_shape=[
            jax.ShapeDtypeStruct((Mp, D), jnp.float32),
            jax.ShapeDtypeStruct((1, D), jnp.float32),
        ],
        scratch_shapes=[pltpu.VMEM((1, D), jnp.float32)],
    )(agg2P, degP, h_unc, cs_own, cs_src, W1_own, W1_src, W2)


def _final(h2r, h2c, cs2_r, cs2_c, W2_i, W2_c):
    B = 400
    nb = N_CELL_C // B

    def body(hr_ref, hc_ref, csr_ref, csc_ref, wr_ref, wc_ref, o_ref):
        i = pl.program_id(0)
        c2r = jnp.dot(csr_ref[...] * (1.0 / N_RID_C), wr_ref[...],
                      preferred_element_type=jnp.float32)
        c2c = jnp.dot(csc_ref[...] * (1.0 / N_CELL_C), wc_ref[...],
                      preferred_element_type=jnp.float32)

        @pl.when(i < 25)
        def _():
            o_ref[...] = jnp.maximum(hr_ref[...] - c2r, 0.0)

        @pl.when(i >= 25)
        def _():
            o_ref[...] = jnp.maximum(hc_ref[...] - c2c, 0.0)

    return pl.pallas_call(
        body,
        grid=(nb,),
        in_specs=[
            pl.BlockSpec((B, D), lambda i: (jnp.minimum(i, 24), 0)),
            pl.BlockSpec((B, D), lambda i: (i, 0)),
            pl.BlockSpec((1, D), lambda i: (0, 0)),
            pl.BlockSpec((1, D), lambda i: (0, 0)),
            pl.BlockSpec((D, D), lambda i: (0, 0)),
            pl.BlockSpec((D, D), lambda i: (0, 0)),
        ],
        out_specs=pl.BlockSpec((B, D), lambda i: (i, 0)),
        out_shape=jax.ShapeDtypeStruct((N_CELL_C, D), jnp.float32),
    )(h2r, h2c, cs2_r, cs2_c, W2_i, W2_c)


def kernel(x_rid, x_cell, src_fwd, dst_fwd, src_bwd, dst_bwd,
           W1_c, b1_c, W1_i, b1_i, W2_c, b2_c, W2_i, b2_i):
    f32 = jnp.float32
    pad_e = E_PAD - E_C
    srcf = jnp.concatenate([src_fwd, jnp.zeros((pad_e,), jnp.int32)])
    dstf = jnp.concatenate([dst_fwd, jnp.full((pad_e,), MP_CELL, jnp.int32)])
    srcb = jnp.concatenate([src_bwd, jnp.zeros((pad_e,), jnp.int32)])
    dstb = jnp.concatenate([dst_bwd, jnp.full((pad_e,), MP_RID, jnp.int32)])
    x_cell_p = jnp.concatenate(
        [x_cell, jnp.zeros((MP_CELL - N_CELL_C, D), f32)])
    x_rid_p = jnp.concatenate(
        [x_rid, jnp.zeros((MP_RID - N_RID_C, D), f32)])

    degc_w = _sc_deg(srcf, dstf, Q_CELL, NP_CELL)
    degr_w = _sc_deg(srcb, dstb, Q_RID, NP_RID)
    degc = degc_w[:, :, 0:1]
    degr = degr_w[:, :, 0:1]

    aggc_p = _sc_agg(x_rid, srcf, dstf, Q_CELL, NP_CELL)
    aggr_p = _sc_agg(x_cell, srcb, dstb, Q_RID, NP_RID)

    h_c, cs_c = _dense1(aggc_p, degc, x_cell_p, W1_c)
    h_r, cs_r = _dense1(aggr_p, degr, x_rid_p, W1_i)

    agg2c_p = _sc_agg(h_r, srcf, dstf, Q_CELL, NP_CELL)
    agg2r_p = _sc_agg(h_c, srcb, dstb, Q_RID, NP_RID)

    h2c, cs2_c = _dense2(agg2c_p, degc, h_c, cs_c, cs_r,
                         W1_c, W1_i, W2_c, N_CELL_C, N_RID_C)
    h2r, cs2_r = _dense2(agg2r_p, degr, h_r, cs_r, cs_c,
                         W1_i, W1_c, W2_i, N_RID_C, N_CELL_C)

    return _final(h2r, h2c, cs2_r, cs2_c, W2_i, W2_c)
